# baseline (speedup 1.0000x reference)
import functools

import jax
import jax.numpy as jnp
from jax import lax
from jax.experimental import pallas as pl
from jax.experimental.pallas import tpu as pltpu
from jax.experimental.pallas import tpu_sc as plsc

_NC = 2
_NS = 16
_NW = _NC * _NS
_CHUNK = 128


def _transform_body(D, tok_ref, path_ref, w_ref, t1_ref, p2_ref, t3_ref):
    f32 = jnp.float32
    h = D // 2

    def rne_hi(bits):
        return (bits + jnp.uint32(0x7FFF) + ((bits >> 16) & jnp.uint32(1))) \
            & jnp.uint32(0xFFFF0000)

    def pack(m):
        lo = rne_hi(pltpu.bitcast(m[:, 0:h], jnp.uint32)) >> 16
        hi = rne_hi(pltpu.bitcast(m[:, h:D], jnp.uint32))
        return hi | lo

    tok = tok_ref[...]
    pat = path_ref[...]
    t1_ref[...] = pack(jnp.dot(tok, w_ref[0:D, :], preferred_element_type=f32))
    p2_ref[...] = pack(jnp.dot(pat, w_ref[D:2 * D, :],
                               preferred_element_type=f32))
    t3_ref[...] = pack(jnp.dot(tok, w_ref[2 * D:3 * D, :],
                               preferred_element_type=f32))


def _tc_transform(token_table, path_table, W, Vt=800):
    V, D = token_table.shape
    assert V % Vt == 0
    grid = (V // Vt,)
    tblk = pl.BlockSpec((Vt, D), lambda i: (i, 0))
    oblk = pl.BlockSpec((Vt, D // 2), lambda i: (i, 0))
    out_t = jax.ShapeDtypeStruct((V, D // 2), jnp.uint32)
    return pl.pallas_call(
        functools.partial(_transform_body, D),
        grid=grid,
        in_specs=[tblk, tblk, pl.BlockSpec((3 * D, D), lambda i: (0, 0))],
        out_specs=[oblk, oblk, oblk],
        out_shape=[out_t, out_t, out_t],
    )(token_table, path_table, W)


def _sc_gather_sum(t1, p2, t3, i0, i1, i2):
    (BL,) = i0.shape
    V, Dw = t1.shape
    rows_per_w = BL // _NW
    nchunk = rows_per_w // _CHUNK
    npair = nchunk // 2
    assert rows_per_w * _NW == BL and npair * 2 * _CHUNK == rows_per_w
    ngrp = Dw // 16

    mesh = plsc.VectorSubcoreMesh(
        core_axis_name="c", subcore_axis_name="s",
        num_cores=_NC, num_subcores=_NS)

    idx_t = pltpu.VMEM((_CHUNK,), jnp.int32)
    row_t = pltpu.VMEM((_CHUNK, Dw), jnp.uint32)

    @functools.partial(
        pl.kernel,
        out_type=jax.ShapeDtypeStruct((BL, Dw), jnp.uint32),
        mesh=mesh,
        scratch_types=[
            [idx_t] * 3, [idx_t] * 3,
            [row_t] * 3, [row_t] * 3,
            pltpu.SemaphoreType.DMA, pltpu.SemaphoreType.DMA,
        ],
        compiler_params=pltpu.CompilerParams(use_tc_tiling_on_sc=False),
    )
    def gather_kernel(t1_hbm, p2_hbm, t3_hbm, i0_hbm, i1_hbm, i2_hbm,
                      o_hbm, idx_a, idx_b, rows_a, rows_b, sem_a, sem_b):
        wid = lax.axis_index("s") * _NC + lax.axis_index("c")
        w_base = wid * rows_per_w
        tabs = (t1_hbm, p2_hbm, t3_hbm)
        idxs = (i0_hbm, i1_hbm, i2_hbm)

        def stage(chunk_no, idx_v, rows_v, sem):
            base = w_base + chunk_no * _CHUNK
            for t in range(3):
                pltpu.sync_copy(idxs[t].at[pl.ds(base, _CHUNK)], idx_v[t])
            for t in range(3):
                pltpu.async_copy(tabs[t].at[idx_v[t]], rows_v[t], sem)

        def drain(rows_v, sem):
            for t in range(3):
                pltpu.make_async_copy(tabs[t].at[pl.ds(0, _CHUNK)], rows_v[t],
                                      sem).wait()

        def add_store(chunk_no, rows_v):
            base = w_base + chunk_no * _CHUNK
            r0, r1, r2 = rows_v
            c_hi = jnp.uint32(0xFFFF0000)
            c_rnd = jnp.uint32(0x8000)
            sh = jnp.uint32(16)

            def row_body(r, carry):
                for g in range(ngrp):
                    sl = (r, pl.ds(g * 16, 16))
                    w0, w1, w2 = r0[sl], r1[sl], r2[sl]
                    lo = (lax.bitcast_convert_type(w0 << sh, jnp.float32)
                          + lax.bitcast_convert_type(w1 << sh, jnp.float32)
                          + lax.bitcast_convert_type(w2 << sh, jnp.float32))
                    hi = (lax.bitcast_convert_type(w0 & c_hi, jnp.float32)
                          + lax.bitcast_convert_type(w1 & c_hi, jnp.float32)
                          + lax.bitcast_convert_type(w2 & c_hi, jnp.float32))
                    lo_r = (lax.bitcast_convert_type(lo, jnp.uint32) + c_rnd) >> sh
                    hi_r = (lax.bitcast_convert_type(hi, jnp.uint32) + c_rnd) & c_hi
                    r0[sl] = hi_r | lo_r
                return carry

            lax.fori_loop(0, _CHUNK, row_body, 0)
            pltpu.sync_copy(r0, o_hbm.at[pl.ds(base, _CHUNK)])

        stage(0, idx_a, rows_a, sem_a)

        def pair_body(j, carry):
            stage(2 * j + 1, idx_b, rows_b, sem_b)
            drain(rows_a, sem_a)
            add_store(2 * j, rows_a)

            @pl.when(j + 1 < npair)
            def _():
                stage(2 * j + 2, idx_a, rows_a, sem_a)

            drain(rows_b, sem_b)
            add_store(2 * j + 1, rows_b)
            return carry

        lax.fori_loop(0, npair, pair_body, 0)

    return gather_kernel(t1, p2, t3, i0, i1, i2)


def _tail_body(L, Dw, s_ref, bl_ref, bh_ref, al_ref, ah_ref, ol_ref, oh_ref):
    w = s_ref[...]
    lo = pltpu.bitcast(w << jnp.uint32(16), jnp.float32)
    hi = pltpu.bitcast(w & jnp.uint32(0xFFFF0000), jnp.float32)
    ct_lo = jnp.tanh(lo + bl_ref[...].reshape(1, 1, Dw))
    ct_hi = jnp.tanh(hi + bh_ref[...].reshape(1, 1, Dw))
    a = jnp.sum(ct_lo * al_ref[...].reshape(1, 1, Dw)
                + ct_hi * ah_ref[...].reshape(1, 1, Dw),
                axis=2, keepdims=True)
    m = jnp.max(a, axis=1, keepdims=True)
    e = jnp.exp(a - m)
    p = e / jnp.sum(e, axis=1, keepdims=True)
    ol_ref[...] = jnp.sum(ct_lo * p, axis=1)
    oh_ref[...] = jnp.sum(ct_hi * p, axis=1)


def _tc_tail(s3, b, ap, Bt=8):
    B, L, Dw = s3.shape
    grid = (B // Bt,)
    vblk = pl.BlockSpec((1, Dw), lambda i: (0, 0))
    out_t = jax.ShapeDtypeStruct((B, Dw), jnp.float32)
    oblk = pl.BlockSpec((Bt, Dw), lambda i: (i, 0))
    b_lo = b[0:Dw].reshape(1, Dw)
    b_hi = b[Dw:2 * Dw].reshape(1, Dw)
    a_lo = ap[0:Dw].reshape(1, Dw)
    a_hi = ap[Dw:2 * Dw].reshape(1, Dw)
    return pl.pallas_call(
        functools.partial(_tail_body, L, Dw),
        grid=grid,
        in_specs=[
            pl.BlockSpec((Bt, L, Dw), lambda i: (i, 0, 0)),
            vblk, vblk, vblk, vblk,
        ],
        out_specs=[oblk, oblk],
        out_shape=[out_t, out_t],
    )(s3, b_lo, b_hi, a_lo, a_hi)


def kernel(x, token_table, path_table, attn_param, W, b):
    B, L, _ = x.shape
    V, D = token_table.shape
    BL = B * L
    xf = x.reshape(BL, 3)
    i0 = xf[:, 0]
    i1 = xf[:, 1]
    i2 = xf[:, 2]
    t1, p2, t3 = _tc_transform(token_table, path_table, W)
    s = _sc_gather_sum(t1, p2, t3, i0, i1, i2)
    v_lo, v_hi = _tc_tail(s.reshape(B, L, D // 2), b, attn_param.reshape(D))
    return jnp.concatenate([v_lo, v_hi], axis=1)


# speedup vs baseline: 1.4565x; 1.4565x over previous
import functools

import jax
import jax.numpy as jnp
from jax import lax
from jax.experimental import pallas as pl
from jax.experimental.pallas import tpu as pltpu
from jax.experimental.pallas import tpu_sc as plsc

_NC = 2
_NS = 16
_NW = _NC * _NS
_CHUNK = 128


def _transform_body(D, ta_ref, tb_ref, pa_ref, pb_ref, w_ref,
                    t1_ref, p2_ref, t3_ref):
    f32 = jnp.float32
    h = D // 2

    def rne_hi(bits):
        return (bits + jnp.uint32(0x7FFF) + ((bits >> 16) & jnp.uint32(1))) \
            & jnp.uint32(0xFFFF0000)

    def pack(m):
        lo = rne_hi(pltpu.bitcast(m[:, 0:h], jnp.uint32)) >> 16
        hi = rne_hi(pltpu.bitcast(m[:, h:D], jnp.uint32))
        return hi | lo

    def both(xa, xb, wslice):
        ya = pack(jnp.dot(xa, wslice, preferred_element_type=f32))
        yb = pack(jnp.dot(xb, wslice, preferred_element_type=f32))
        return jnp.concatenate([ya, yb], axis=1)

    ta = ta_ref[...]
    tb = tb_ref[...]
    pa = pa_ref[...]
    pb = pb_ref[...]
    t1_ref[...] = both(ta, tb, w_ref[0:D, :])
    p2_ref[...] = both(pa, pb, w_ref[D:2 * D, :])
    t3_ref[...] = both(ta, tb, w_ref[2 * D:3 * D, :])


def _tc_transform(token_table, path_table, W, Vt2=400):
    V, D = token_table.shape
    half = V // 2
    nb = half // Vt2
    assert nb * Vt2 == half
    grid = (nb,)
    ablk = pl.BlockSpec((Vt2, D), lambda i: (i, 0))
    bblk = pl.BlockSpec((Vt2, D), lambda i: (i + nb, 0))
    oblk = pl.BlockSpec((Vt2, D), lambda i: (i, 0))
    out_t = jax.ShapeDtypeStruct((half, D), jnp.uint32)
    return pl.pallas_call(
        functools.partial(_transform_body, D),
        grid=grid,
        in_specs=[ablk, bblk, ablk, bblk,
                  pl.BlockSpec((3 * D, D), lambda i: (0, 0))],
        out_specs=[oblk, oblk, oblk],
        out_shape=[out_t, out_t, out_t],
    )(token_table, token_table, path_table, path_table, W)


def _sc_gather_sum(t1, p2, t3, i0, i1, i2):
    (BL,) = i0.shape
    V, Dw = t1.shape
    rows_per_w = BL // _NW
    nchunk = rows_per_w // _CHUNK
    npair = nchunk // 2
    assert rows_per_w * _NW == BL and npair * 2 * _CHUNK == rows_per_w
    ngrp = Dw // 16

    mesh = plsc.VectorSubcoreMesh(
        core_axis_name="c", subcore_axis_name="s",
        num_cores=_NC, num_subcores=_NS)

    idx_t = pltpu.VMEM((_CHUNK,), jnp.int32)
    row_t = pltpu.VMEM((_CHUNK, Dw), jnp.uint32)
    st_t = pltpu.VMEM((_CHUNK // 2, 2 * Dw), jnp.uint32)

    @functools.partial(
        pl.kernel,
        out_type=jax.ShapeDtypeStruct((BL // 2, 2 * Dw), jnp.uint32),
        mesh=mesh,
        scratch_types=[
            [idx_t] * 3, [idx_t] * 3,
            [row_t] * 3, [row_t] * 3,
            st_t, st_t,
            pltpu.SemaphoreType.DMA, pltpu.SemaphoreType.DMA,
        ],
        compiler_params=pltpu.CompilerParams(use_tc_tiling_on_sc=False),
    )
    def gather_kernel(t1_hbm, p2_hbm, t3_hbm, i0_hbm, i1_hbm, i2_hbm,
                      o_hbm, idx_a, idx_b, rows_a, rows_b, sv_a, sv_b,
                      sem_a, sem_b):
        wid = lax.axis_index("s") * _NC + lax.axis_index("c")
        w_base = wid * rows_per_w
        tabs = (t1_hbm, p2_hbm, t3_hbm)
        idxs = (i0_hbm, i1_hbm, i2_hbm)

        def stage(chunk_no, idx_v, rows_v, sem):
            base = w_base + chunk_no * _CHUNK
            for t in range(3):
                pltpu.sync_copy(idxs[t].at[pl.ds(base, _CHUNK)], idx_v[t])
            for t in range(3):
                pltpu.async_copy(tabs[t].at[idx_v[t]], rows_v[t], sem)

        def drain(rows_v, sem):
            for t in range(3):
                pltpu.make_async_copy(tabs[t].at[pl.ds(0, _CHUNK)], rows_v[t],
                                      sem).wait()

        def add_store(chunk_no, rows_v, s_v):
            base = w_base + chunk_no * _CHUNK
            r0, r1, r2 = rows_v
            c_hi = jnp.uint32(0xFFFF0000)
            c_rnd = jnp.uint32(0x8000)
            sh = jnp.uint32(16)

            def pair_rows(rp, carry):
                for half in range(2):
                    r = rp * 2 + half
                    for g in range(ngrp):
                        sl = (r, pl.ds(g * 16, 16))
                        w0, w1, w2 = r0[sl], r1[sl], r2[sl]
                        lo = (lax.bitcast_convert_type(w0 << sh, jnp.float32)
                              + lax.bitcast_convert_type(w1 << sh, jnp.float32)
                              + lax.bitcast_convert_type(w2 << sh, jnp.float32))
                        hi = (lax.bitcast_convert_type(w0 & c_hi, jnp.float32)
                              + lax.bitcast_convert_type(w1 & c_hi, jnp.float32)
                              + lax.bitcast_convert_type(w2 & c_hi, jnp.float32))
                        lo_r = (lax.bitcast_convert_type(lo, jnp.uint32)
                                + c_rnd) >> sh
                        hi_r = (lax.bitcast_convert_type(hi, jnp.uint32)
                                + c_rnd) & c_hi
                        s_v[rp, pl.ds(half * Dw + g * 16, 16)] = hi_r | lo_r
                return carry

            lax.fori_loop(0, _CHUNK // 2, pair_rows, 0)
            pltpu.sync_copy(s_v, o_hbm.at[pl.ds(base // 2, _CHUNK // 2)])

        stage(0, idx_a, rows_a, sem_a)

        def pair_body(j, carry):
            stage(2 * j + 1, idx_b, rows_b, sem_b)
            drain(rows_a, sem_a)
            add_store(2 * j, rows_a, sv_a)

            @pl.when(j + 1 < npair)
            def _():
                stage(2 * j + 2, idx_a, rows_a, sem_a)

            drain(rows_b, sem_b)
            add_store(2 * j + 1, rows_b, sv_b)
            return carry

        lax.fori_loop(0, npair, pair_body, 0)

    return gather_kernel(t1, p2, t3, i0, i1, i2)


def _tail_body(Bt, Lh, D, s_ref, ba_ref, bb_ref, aa_ref, ab_ref, o_ref):
    h = D // 2
    w3 = s_ref[...].reshape(Bt, Lh, D)
    A = pltpu.bitcast(w3 << jnp.uint32(16), jnp.float32)
    Bm = pltpu.bitcast(w3 & jnp.uint32(0xFFFF0000), jnp.float32)
    ct_a = jnp.tanh(A + ba_ref[...].reshape(1, 1, D))
    ct_b = jnp.tanh(Bm + bb_ref[...].reshape(1, 1, D))
    z = ct_a * aa_ref[...].reshape(1, 1, D) + ct_b * ab_ref[...].reshape(1, 1, D)
    le = jnp.sum(z[:, :, 0:h], axis=2, keepdims=True)
    lo = jnp.sum(z[:, :, h:D], axis=2, keepdims=True)
    m = jnp.maximum(jnp.max(le, axis=1, keepdims=True),
                    jnp.max(lo, axis=1, keepdims=True))
    ee = jnp.exp(le - m)
    eo = jnp.exp(lo - m)
    zsum = jnp.sum(ee, axis=1, keepdims=True) + jnp.sum(eo, axis=1,
                                                        keepdims=True)
    pe = ee / zsum
    po = eo / zsum
    pw = jnp.concatenate([jnp.broadcast_to(pe, (Bt, Lh, h)),
                          jnp.broadcast_to(po, (Bt, Lh, h))], axis=2)
    va = jnp.sum(ct_a * pw, axis=1)
    vb = jnp.sum(ct_b * pw, axis=1)
    o_ref[...] = jnp.concatenate([va[:, 0:h] + va[:, h:D],
                                  vb[:, 0:h] + vb[:, h:D]], axis=1)


def _tc_tail(s2, b, ap, B, L, D, Bt=8):
    Lh = L // 2
    grid = (B // Bt,)
    vblk = pl.BlockSpec((1, D), lambda i: (0, 0))
    h = D // 2
    b0, b1 = b[0:h], b[h:D]
    a0, a1 = ap[0:h], ap[h:D]
    ba = jnp.concatenate([b0, b0]).reshape(1, D)
    bb = jnp.concatenate([b1, b1]).reshape(1, D)
    aa = jnp.concatenate([a0, a0]).reshape(1, D)
    ab = jnp.concatenate([a1, a1]).reshape(1, D)
    return pl.pallas_call(
        functools.partial(_tail_body, Bt, Lh, D),
        grid=grid,
        in_specs=[
            pl.BlockSpec((Bt * Lh, D), lambda i: (i, 0)),
            vblk, vblk, vblk, vblk,
        ],
        out_specs=pl.BlockSpec((Bt, D), lambda i: (i, 0)),
        out_shape=jax.ShapeDtypeStruct((B, D), jnp.float32),
    )(s2, ba, bb, aa, ab)


def kernel(x, token_table, path_table, attn_param, W, b):
    B, L, _ = x.shape
    V, D = token_table.shape
    BL = B * L
    xf = x.reshape(BL, 3)
    half = V // 2
    xr = jnp.where(xf >= half, 2 * (xf - half) + 1, 2 * xf)
    i0 = xr[:, 0]
    i1 = xr[:, 1]
    i2 = xr[:, 2]
    t1, p2, t3 = _tc_transform(token_table, path_table, W)
    t1v = t1.reshape(V, D // 2)
    p2v = p2.reshape(V, D // 2)
    t3v = t3.reshape(V, D // 2)
    s = _sc_gather_sum(t1v, p2v, t3v, i0, i1, i2)
    return _tc_tail(s, b, attn_param.reshape(D), B, L, D)


# speedup vs baseline: 1.6159x; 1.1094x over previous
import functools

import jax
import jax.numpy as jnp
from jax import lax
from jax.experimental import pallas as pl
from jax.experimental.pallas import tpu as pltpu
from jax.experimental.pallas import tpu_sc as plsc

_NC = 2
_NS = 16
_NW = _NC * _NS
_CHUNK = 128


def _transform_body(D, ta_ref, tb_ref, pa_ref, pb_ref, w_ref,
                    t1_ref, p2_ref, t3_ref):
    f32 = jnp.float32
    h = D // 2

    c_rnd = jnp.uint32(0x8000)

    def pack(m):
        lo = (pltpu.bitcast(m[:, 0:h], jnp.uint32) + c_rnd) >> 16
        hi = (pltpu.bitcast(m[:, h:D], jnp.uint32) + c_rnd) \
            & jnp.uint32(0xFFFF0000)
        return hi | lo

    def both(xa, xb, wslice):
        ya = pack(jnp.dot(xa, wslice, preferred_element_type=f32))
        yb = pack(jnp.dot(xb, wslice, preferred_element_type=f32))
        return jnp.concatenate([ya, yb], axis=1)

    ta = ta_ref[...]
    tb = tb_ref[...]
    pa = pa_ref[...]
    pb = pb_ref[...]
    t1_ref[...] = both(ta, tb, w_ref[0:D, :])
    p2_ref[...] = both(pa, pb, w_ref[D:2 * D, :])
    t3_ref[...] = both(ta, tb, w_ref[2 * D:3 * D, :])


def _tc_transform(token_table, path_table, W, Vt2=400):
    V, D = token_table.shape
    half = V // 2
    nb = half // Vt2
    assert nb * Vt2 == half
    grid = (nb,)
    ablk = pl.BlockSpec((Vt2, D), lambda i: (i, 0))
    bblk = pl.BlockSpec((Vt2, D), lambda i: (i + nb, 0))
    oblk = pl.BlockSpec((Vt2, D), lambda i: (i, 0))
    out_t = jax.ShapeDtypeStruct((half, D), jnp.uint32)
    return pl.pallas_call(
        functools.partial(_transform_body, D),
        grid=grid,
        in_specs=[ablk, bblk, ablk, bblk,
                  pl.BlockSpec((3 * D, D), lambda i: (0, 0))],
        out_specs=[oblk, oblk, oblk],
        out_shape=[out_t, out_t, out_t],
    )(token_table, token_table, path_table, path_table, W)


def _sc_gather_sum(t1, p2, t3, i0, i1, i2):
    (BL,) = i0.shape
    V, Dw = t1.shape
    rows_per_w = BL // _NW
    nchunk = rows_per_w // _CHUNK
    npair = nchunk // 2
    assert rows_per_w * _NW == BL and npair * 2 * _CHUNK == rows_per_w
    ngrp = Dw // 16

    mesh = plsc.VectorSubcoreMesh(
        core_axis_name="c", subcore_axis_name="s",
        num_cores=_NC, num_subcores=_NS)

    idx_t = pltpu.VMEM((_CHUNK,), jnp.int32)
    row_t = pltpu.VMEM((_CHUNK, Dw), jnp.uint32)
    st_t = pltpu.VMEM((_CHUNK, 2 * Dw), jnp.uint32)

    @functools.partial(
        pl.kernel,
        out_type=jax.ShapeDtypeStruct((BL // 2, 2 * Dw), jnp.uint32),
        mesh=mesh,
        scratch_types=[
            [idx_t] * 3, [idx_t] * 3,
            [row_t] * 3, [row_t] * 3,
            st_t,
            pltpu.SemaphoreType.DMA, pltpu.SemaphoreType.DMA,
            pltpu.SemaphoreType.DMA,
        ],
        compiler_params=pltpu.CompilerParams(use_tc_tiling_on_sc=False),
    )
    def gather_kernel(t1_hbm, p2_hbm, t3_hbm, i0_hbm, i1_hbm, i2_hbm,
                      o_hbm, idx_a, idx_b, rows_a, rows_b, s_v,
                      sem_a, sem_b, sem_st):
        wid = lax.axis_index("s") * _NC + lax.axis_index("c")
        w_base = wid * rows_per_w
        tabs = (t1_hbm, p2_hbm, t3_hbm)
        idxs = (i0_hbm, i1_hbm, i2_hbm)

        def stage(chunk_no, idx_v, rows_v, sem):
            base = w_base + chunk_no * _CHUNK
            for t in range(3):
                pltpu.sync_copy(idxs[t].at[pl.ds(base, _CHUNK)], idx_v[t])
            for t in range(3):
                pltpu.async_copy(tabs[t].at[idx_v[t]], rows_v[t], sem)

        def drain(rows_v, sem):
            for t in range(3):
                pltpu.make_async_copy(tabs[t].at[pl.ds(0, _CHUNK)], rows_v[t],
                                      sem).wait()

        def drain_store():
            pltpu.make_async_copy(s_v, o_hbm.at[pl.ds(0, _CHUNK)],
                                  sem_st).wait()

        def add(rows_v, s_off):
            r0, r1, r2 = rows_v
            c_hi = jnp.uint32(0xFFFF0000)
            c_rnd = jnp.uint32(0x8000)
            sh = jnp.uint32(16)

            @plsc.parallel_loop(0, _CHUNK // 2)
            def pair_rows(rp):
                for half in range(2):
                    r = rp * 2 + half
                    for g in range(ngrp):
                        sl = (r, pl.ds(g * 16, 16))
                        w0, w1, w2 = r0[sl], r1[sl], r2[sl]
                        lo = (lax.bitcast_convert_type(w0 << sh, jnp.float32)
                              + lax.bitcast_convert_type(w1 << sh, jnp.float32)
                              + lax.bitcast_convert_type(w2 << sh, jnp.float32))
                        hi = (lax.bitcast_convert_type(w0 & c_hi, jnp.float32)
                              + lax.bitcast_convert_type(w1 & c_hi, jnp.float32)
                              + lax.bitcast_convert_type(w2 & c_hi, jnp.float32))
                        lo_r = (lax.bitcast_convert_type(lo, jnp.uint32)
                                + c_rnd) >> sh
                        hi_r = (lax.bitcast_convert_type(hi, jnp.uint32)
                                + c_rnd) & c_hi
                        s_v[s_off + rp, pl.ds(half * Dw + g * 16, 16)] = \
                            hi_r | lo_r

        stage(0, idx_a, rows_a, sem_a)

        def pair_body(j, carry):
            stage(2 * j + 1, idx_b, rows_b, sem_b)
            drain(rows_a, sem_a)

            @pl.when(j > 0)
            def _():
                drain_store()

            add(rows_a, 0)

            @pl.when(j + 1 < npair)
            def _():
                stage(2 * j + 2, idx_a, rows_a, sem_a)

            drain(rows_b, sem_b)
            add(rows_b, _CHUNK // 2)
            pltpu.async_copy(
                s_v, o_hbm.at[pl.ds(w_base // 2 + j * _CHUNK, _CHUNK)],
                sem_st)
            return carry

        lax.fori_loop(0, npair, pair_body, 0)
        drain_store()

    return gather_kernel(t1, p2, t3, i0, i1, i2)


def _tail_body(Bt, Lh, D, s_ref, ba_ref, bb_ref, aa_ref, ab_ref, o_ref):
    h = D // 2
    w3 = s_ref[...].reshape(Bt, Lh, D)
    A = pltpu.bitcast(w3 << jnp.uint32(16), jnp.float32)
    Bm = pltpu.bitcast(w3 & jnp.uint32(0xFFFF0000), jnp.float32)
    ct_a = jnp.tanh(A + ba_ref[...].reshape(1, 1, D))
    ct_b = jnp.tanh(Bm + bb_ref[...].reshape(1, 1, D))
    z = ct_a * aa_ref[...].reshape(1, 1, D) + ct_b * ab_ref[...].reshape(1, 1, D)
    le = jnp.sum(z[:, :, 0:h], axis=2, keepdims=True)
    lo = jnp.sum(z[:, :, h:D], axis=2, keepdims=True)
    m = jnp.maximum(jnp.max(le, axis=1, keepdims=True),
                    jnp.max(lo, axis=1, keepdims=True))
    ee = jnp.exp(le - m)
    eo = jnp.exp(lo - m)
    zsum = jnp.sum(ee, axis=1, keepdims=True) + jnp.sum(eo, axis=1,
                                                        keepdims=True)
    pe = ee / zsum
    po = eo / zsum
    pw = jnp.concatenate([jnp.broadcast_to(pe, (Bt, Lh, h)),
                          jnp.broadcast_to(po, (Bt, Lh, h))], axis=2)
    va = jnp.sum(ct_a * pw, axis=1)
    vb = jnp.sum(ct_b * pw, axis=1)
    o_ref[...] = jnp.concatenate([va[:, 0:h] + va[:, h:D],
                                  vb[:, 0:h] + vb[:, h:D]], axis=1)


def _tc_tail(s2, b, ap, B, L, D, Bt=16):
    Lh = L // 2
    grid = (B // Bt,)
    vblk = pl.BlockSpec((1, D), lambda i: (0, 0))
    h = D // 2
    b0, b1 = b[0:h], b[h:D]
    a0, a1 = ap[0:h], ap[h:D]
    ba = jnp.concatenate([b0, b0]).reshape(1, D)
    bb = jnp.concatenate([b1, b1]).reshape(1, D)
    aa = jnp.concatenate([a0, a0]).reshape(1, D)
    ab = jnp.concatenate([a1, a1]).reshape(1, D)
    return pl.pallas_call(
        functools.partial(_tail_body, Bt, Lh, D),
        grid=grid,
        in_specs=[
            pl.BlockSpec((Bt * Lh, D), lambda i: (i, 0)),
            vblk, vblk, vblk, vblk,
        ],
        out_specs=pl.BlockSpec((Bt, D), lambda i: (i, 0)),
        out_shape=jax.ShapeDtypeStruct((B, D), jnp.float32),
    )(s2, ba, bb, aa, ab)


def kernel(x, token_table, path_table, attn_param, W, b):
    B, L, _ = x.shape
    V, D = token_table.shape
    BL = B * L
    xf = x.reshape(BL, 3)
    half = V // 2
    xr = jnp.where(xf >= half, 2 * (xf - half) + 1, 2 * xf)
    i0 = xr[:, 0]
    i1 = xr[:, 1]
    i2 = xr[:, 2]
    t1, p2, t3 = _tc_transform(token_table, path_table, W)
    t1v = t1.reshape(V, D // 2)
    p2v = p2.reshape(V, D // 2)
    t3v = t3.reshape(V, D // 2)
    s = _sc_gather_sum(t1v, p2v, t3v, i0, i1, i2)
    return _tc_tail(s, b, attn_param.reshape(D), B, L, D)


# speedup vs baseline: 1.6302x; 1.0089x over previous
import functools

import jax
import jax.numpy as jnp
from jax import lax
from jax.experimental import pallas as pl
from jax.experimental.pallas import tpu as pltpu
from jax.experimental.pallas import tpu_sc as plsc

_NC = 2
_NS = 16
_NW = _NC * _NS
_CHUNK = 128


def _transform_body(D, ta_ref, tb_ref, pa_ref, pb_ref, w_ref,
                    t1_ref, p2_ref, t3_ref):
    f32 = jnp.float32
    h = D // 2

    c_rnd = jnp.uint32(0x8000)

    def pack(m):
        lo = (pltpu.bitcast(m[:, 0:h], jnp.uint32) + c_rnd) >> 16
        hi = (pltpu.bitcast(m[:, h:D], jnp.uint32) + c_rnd) \
            & jnp.uint32(0xFFFF0000)
        return hi | lo

    def both(xa, xb, wslice):
        ya = pack(jnp.dot(xa, wslice, preferred_element_type=f32))
        yb = pack(jnp.dot(xb, wslice, preferred_element_type=f32))
        return jnp.concatenate([ya, yb], axis=1)

    ta = ta_ref[...]
    tb = tb_ref[...]
    pa = pa_ref[...]
    pb = pb_ref[...]
    t1_ref[...] = both(ta, tb, w_ref[0:D, :])
    p2_ref[...] = both(pa, pb, w_ref[D:2 * D, :])
    t3_ref[...] = both(ta, tb, w_ref[2 * D:3 * D, :])


def _tc_transform(token_table, path_table, W, Vt2=400):
    V, D = token_table.shape
    half = V // 2
    nb = half // Vt2
    assert nb * Vt2 == half
    grid = (nb,)
    ablk = pl.BlockSpec((Vt2, D), lambda i: (i, 0))
    bblk = pl.BlockSpec((Vt2, D), lambda i: (i + nb, 0))
    oblk = pl.BlockSpec((Vt2, D), lambda i: (i, 0))
    out_t = jax.ShapeDtypeStruct((half, D), jnp.uint32)
    return pl.pallas_call(
        functools.partial(_transform_body, D),
        grid=grid,
        in_specs=[ablk, bblk, ablk, bblk,
                  pl.BlockSpec((3 * D, D), lambda i: (0, 0))],
        out_specs=[oblk, oblk, oblk],
        out_shape=[out_t, out_t, out_t],
    )(token_table, token_table, path_table, path_table, W)


def _sc_gather_sum(t1, p2, t3, i0, i1, i2, chunk=_CHUNK):
    (BL,) = i0.shape
    V, Dw = t1.shape
    rows_per_w = BL // _NW
    nchunk = rows_per_w // chunk
    npair = nchunk // 2
    assert rows_per_w * _NW == BL and npair * 2 * chunk == rows_per_w
    ngrp = Dw // 16

    mesh = plsc.VectorSubcoreMesh(
        core_axis_name="c", subcore_axis_name="s",
        num_cores=_NC, num_subcores=_NS)

    idx_t = pltpu.VMEM((chunk,), jnp.int32)
    row_t = pltpu.VMEM((chunk, Dw), jnp.uint32)
    st_t = pltpu.VMEM((chunk, 2 * Dw), jnp.uint32)

    @functools.partial(
        pl.kernel,
        out_type=jax.ShapeDtypeStruct((BL // 2, 2 * Dw), jnp.uint32),
        mesh=mesh,
        scratch_types=[
            [idx_t] * 3, [idx_t] * 3,
            [row_t] * 3, [row_t] * 3,
            st_t,
            pltpu.SemaphoreType.DMA, pltpu.SemaphoreType.DMA,
            pltpu.SemaphoreType.DMA,
        ],
        compiler_params=pltpu.CompilerParams(use_tc_tiling_on_sc=False),
    )
    def gather_kernel(t1_hbm, p2_hbm, t3_hbm, i0_hbm, i1_hbm, i2_hbm,
                      o_hbm, idx_a, idx_b, rows_a, rows_b, s_v,
                      sem_a, sem_b, sem_st):
        wid = lax.axis_index("s") * _NC + lax.axis_index("c")
        w_base = wid * rows_per_w
        tabs = (t1_hbm, p2_hbm, t3_hbm)
        idxs = (i0_hbm, i1_hbm, i2_hbm)

        def stage(chunk_no, idx_v, rows_v, sem):
            base = w_base + chunk_no * chunk
            for t in range(3):
                pltpu.sync_copy(idxs[t].at[pl.ds(base, chunk)], idx_v[t])
            for t in range(3):
                pltpu.async_copy(tabs[t].at[idx_v[t]], rows_v[t], sem)

        def drain(rows_v, sem):
            for t in range(3):
                pltpu.make_async_copy(tabs[t].at[pl.ds(0, chunk)], rows_v[t],
                                      sem).wait()

        def drain_store():
            pltpu.make_async_copy(s_v, o_hbm.at[pl.ds(0, chunk)],
                                  sem_st).wait()

        def add(rows_v, s_off):
            r0, r1, r2 = rows_v
            c_hi = jnp.uint32(0xFFFF0000)
            c_rnd = jnp.uint32(0x8000)
            sh = jnp.uint32(16)

            @plsc.parallel_loop(0, chunk // 2)
            def pair_rows(rp):
                for half in range(2):
                    r = rp * 2 + half
                    for g in range(ngrp):
                        sl = (r, pl.ds(g * 16, 16))
                        w0, w1, w2 = r0[sl], r1[sl], r2[sl]
                        lo = (lax.bitcast_convert_type(w0 << sh, jnp.float32)
                              + lax.bitcast_convert_type(w1 << sh, jnp.float32)
                              + lax.bitcast_convert_type(w2 << sh, jnp.float32))
                        hi = (lax.bitcast_convert_type(w0 & c_hi, jnp.float32)
                              + lax.bitcast_convert_type(w1 & c_hi, jnp.float32)
                              + lax.bitcast_convert_type(w2 & c_hi, jnp.float32))
                        lo_r = (lax.bitcast_convert_type(lo, jnp.uint32)
                                + c_rnd) >> sh
                        hi_r = (lax.bitcast_convert_type(hi, jnp.uint32)
                                + c_rnd) & c_hi
                        s_v[s_off + rp, pl.ds(half * Dw + g * 16, 16)] = \
                            hi_r | lo_r

        stage(0, idx_a, rows_a, sem_a)

        def pair_body(j, carry):
            stage(2 * j + 1, idx_b, rows_b, sem_b)
            drain(rows_a, sem_a)

            @pl.when(j > 0)
            def _():
                drain_store()

            add(rows_a, 0)

            @pl.when(j + 1 < npair)
            def _():
                stage(2 * j + 2, idx_a, rows_a, sem_a)

            drain(rows_b, sem_b)
            add(rows_b, chunk // 2)
            pltpu.async_copy(
                s_v, o_hbm.at[pl.ds(w_base // 2 + j * chunk, chunk)],
                sem_st)
            return carry

        lax.fori_loop(0, npair, pair_body, 0)
        drain_store()

    return gather_kernel(t1, p2, t3, i0, i1, i2)


def _tail_body(Bt, Lh, D, s_ref, ba_ref, bb_ref, aa_ref, ab_ref, o_ref):
    h = D // 2
    w3 = s_ref[...].reshape(Bt, Lh, D)
    A = pltpu.bitcast(w3 << jnp.uint32(16), jnp.float32)
    Bm = pltpu.bitcast(w3 & jnp.uint32(0xFFFF0000), jnp.float32)
    ct_a = jnp.tanh(A + ba_ref[...].reshape(1, 1, D))
    ct_b = jnp.tanh(Bm + bb_ref[...].reshape(1, 1, D))
    z = ct_a * aa_ref[...].reshape(1, 1, D) + ct_b * ab_ref[...].reshape(1, 1, D)
    le = jnp.sum(z[:, :, 0:h], axis=2, keepdims=True)
    lo = jnp.sum(z[:, :, h:D], axis=2, keepdims=True)
    m = jnp.maximum(jnp.max(le, axis=1, keepdims=True),
                    jnp.max(lo, axis=1, keepdims=True))
    ee = jnp.exp(le - m)
    eo = jnp.exp(lo - m)
    zsum = jnp.sum(ee, axis=1, keepdims=True) + jnp.sum(eo, axis=1,
                                                        keepdims=True)
    pe = ee / zsum
    po = eo / zsum
    pw = jnp.concatenate([jnp.broadcast_to(pe, (Bt, Lh, h)),
                          jnp.broadcast_to(po, (Bt, Lh, h))], axis=2)
    va = jnp.sum(ct_a * pw, axis=1)
    vb = jnp.sum(ct_b * pw, axis=1)
    o_ref[...] = jnp.concatenate([va[:, 0:h] + va[:, h:D],
                                  vb[:, 0:h] + vb[:, h:D]], axis=1)


def _tc_tail(s2, b, ap, B, L, D, Bt=16):
    Lh = L // 2
    grid = (B // Bt,)
    vblk = pl.BlockSpec((1, D), lambda i: (0, 0))
    h = D // 2
    b0, b1 = b[0:h], b[h:D]
    a0, a1 = ap[0:h], ap[h:D]
    ba = jnp.concatenate([b0, b0]).reshape(1, D)
    bb = jnp.concatenate([b1, b1]).reshape(1, D)
    aa = jnp.concatenate([a0, a0]).reshape(1, D)
    ab = jnp.concatenate([a1, a1]).reshape(1, D)
    return pl.pallas_call(
        functools.partial(_tail_body, Bt, Lh, D),
        grid=grid,
        in_specs=[
            pl.BlockSpec((Bt * Lh, D), lambda i: (i, 0)),
            vblk, vblk, vblk, vblk,
        ],
        out_specs=pl.BlockSpec((Bt, D), lambda i: (i, 0)),
        out_shape=jax.ShapeDtypeStruct((B, D), jnp.float32),
    )(s2, ba, bb, aa, ab)


def kernel(x, token_table, path_table, attn_param, W, b):
    B, L, _ = x.shape
    V, D = token_table.shape
    BL = B * L
    xf = x.reshape(BL, 3)
    half = V // 2
    xr = jnp.where(xf >= half, 2 * (xf - half) + 1, 2 * xf)
    t1, p2, t3 = _tc_transform(token_table, path_table, W)
    t1v = t1.reshape(V, D // 2)
    p2v = p2.reshape(V, D // 2)
    t3v = t3.reshape(V, D // 2)
    Bh = B // 2
    BLh = BL // 2
    ap = attn_param.reshape(D)
    outs = []
    for hx in range(2):
        lo = hx * BLh
        s = _sc_gather_sum(t1v, p2v, t3v,
                           xr[lo:lo + BLh, 0], xr[lo:lo + BLh, 1],
                           xr[lo:lo + BLh, 2], chunk=80)
        outs.append(_tc_tail(s, b, ap, Bh, L, D))
    return jnp.concatenate(outs, axis=0)


# speedup vs baseline: 1.7668x; 1.0838x over previous
import functools

import jax
import jax.numpy as jnp
from jax import lax
from jax.experimental import pallas as pl
from jax.experimental.pallas import tpu as pltpu
from jax.experimental.pallas import tpu_sc as plsc

_NC = 2
_NS = 16
_NW = _NC * _NS
_CHUNK = 128


def _transform_body(D, ta_ref, tb_ref, pa_ref, pb_ref, w_ref,
                    t1_ref, p2_ref, t3_ref):
    f32 = jnp.float32
    h = D // 2

    c_rnd = jnp.uint32(0x8000)

    def pack(m):
        lo = (pltpu.bitcast(m[:, 0:h], jnp.uint32) + c_rnd) >> 16
        hi = (pltpu.bitcast(m[:, h:D], jnp.uint32) + c_rnd) \
            & jnp.uint32(0xFFFF0000)
        return hi | lo

    def both(xa, xb, wslice):
        ya = pack(jnp.dot(xa, wslice, preferred_element_type=f32))
        yb = pack(jnp.dot(xb, wslice, preferred_element_type=f32))
        return jnp.concatenate([ya, yb], axis=1)

    ta = ta_ref[...]
    tb = tb_ref[...]
    pa = pa_ref[...]
    pb = pb_ref[...]
    t1_ref[...] = both(ta, tb, w_ref[0:D, :])
    p2_ref[...] = both(pa, pb, w_ref[D:2 * D, :])
    t3_ref[...] = both(ta, tb, w_ref[2 * D:3 * D, :])


def _tc_transform(token_table, path_table, W, Vt2=400):
    V, D = token_table.shape
    half = V // 2
    nb = half // Vt2
    assert nb * Vt2 == half
    grid = (nb,)
    ablk = pl.BlockSpec((Vt2, D), lambda i: (i, 0))
    bblk = pl.BlockSpec((Vt2, D), lambda i: (i + nb, 0))
    oblk = pl.BlockSpec((Vt2, D), lambda i: (i, 0))
    out_t = jax.ShapeDtypeStruct((half, D), jnp.uint32)
    return pl.pallas_call(
        functools.partial(_transform_body, D),
        grid=grid,
        in_specs=[ablk, bblk, ablk, bblk,
                  pl.BlockSpec((3 * D, D), lambda i: (0, 0))],
        out_specs=[oblk, oblk, oblk],
        out_shape=[out_t, out_t, out_t],
    )(token_table, token_table, path_table, path_table, W)


def _sc_gather_sum(t1, p2, t3, i0, i1, i2, chunk=_CHUNK):
    (BL,) = i0.shape
    V, Dw = t1.shape
    rows_per_w = BL // _NW
    nchunk = rows_per_w // chunk
    npair = nchunk // 2
    leftover = nchunk - 2 * npair
    assert rows_per_w * _NW == BL and nchunk * chunk == rows_per_w
    ngrp = Dw // 16

    mesh = plsc.VectorSubcoreMesh(
        core_axis_name="c", subcore_axis_name="s",
        num_cores=_NC, num_subcores=_NS)

    idx_t = pltpu.VMEM((chunk,), jnp.int32)
    row_t = pltpu.VMEM((chunk, Dw), jnp.uint32)
    st_t = pltpu.VMEM((chunk, 2 * Dw), jnp.uint32)

    @functools.partial(
        pl.kernel,
        out_type=jax.ShapeDtypeStruct((BL // 2, 2 * Dw), jnp.uint32),
        mesh=mesh,
        scratch_types=[
            [idx_t] * 3, [idx_t] * 3,
            [row_t] * 3, [row_t] * 3,
            st_t,
            pltpu.SemaphoreType.DMA, pltpu.SemaphoreType.DMA,
            pltpu.SemaphoreType.DMA,
        ],
        compiler_params=pltpu.CompilerParams(use_tc_tiling_on_sc=False),
    )
    def gather_kernel(t1_hbm, p2_hbm, t3_hbm, i0_hbm, i1_hbm, i2_hbm,
                      o_hbm, idx_a, idx_b, rows_a, rows_b, s_v,
                      sem_a, sem_b, sem_st):
        wid = lax.axis_index("s") * _NC + lax.axis_index("c")
        w_base = wid * rows_per_w
        tabs = (t1_hbm, p2_hbm, t3_hbm)
        idxs = (i0_hbm, i1_hbm, i2_hbm)

        def stage(chunk_no, idx_v, rows_v, sem):
            base = w_base + chunk_no * chunk
            for t in range(3):
                pltpu.sync_copy(idxs[t].at[pl.ds(base, chunk)], idx_v[t])
            for t in range(3):
                pltpu.async_copy(tabs[t].at[idx_v[t]], rows_v[t], sem)

        def drain(rows_v, sem):
            for t in range(3):
                pltpu.make_async_copy(tabs[t].at[pl.ds(0, chunk)], rows_v[t],
                                      sem).wait()

        def drain_store():
            pltpu.make_async_copy(s_v, o_hbm.at[pl.ds(0, chunk)],
                                  sem_st).wait()

        def add(rows_v, s_off):
            r0, r1, r2 = rows_v
            c_hi = jnp.uint32(0xFFFF0000)
            c_rnd = jnp.uint32(0x8000)
            sh = jnp.uint32(16)

            @plsc.parallel_loop(0, chunk // 2)
            def pair_rows(rp):
                for half in range(2):
                    r = rp * 2 + half
                    for g in range(ngrp):
                        sl = (r, pl.ds(g * 16, 16))
                        w0, w1, w2 = r0[sl], r1[sl], r2[sl]
                        lo = (lax.bitcast_convert_type(w0 << sh, jnp.float32)
                              + lax.bitcast_convert_type(w1 << sh, jnp.float32)
                              + lax.bitcast_convert_type(w2 << sh, jnp.float32))
                        hi = (lax.bitcast_convert_type(w0 & c_hi, jnp.float32)
                              + lax.bitcast_convert_type(w1 & c_hi, jnp.float32)
                              + lax.bitcast_convert_type(w2 & c_hi, jnp.float32))
                        lo_r = (lax.bitcast_convert_type(lo, jnp.uint32)
                                + c_rnd) >> sh
                        hi_r = (lax.bitcast_convert_type(hi, jnp.uint32)
                                + c_rnd) & c_hi
                        s_v[s_off + rp, pl.ds(half * Dw + g * 16, 16)] = \
                            hi_r | lo_r

        stage(0, idx_a, rows_a, sem_a)

        def pair_body(j, carry):
            stage(2 * j + 1, idx_b, rows_b, sem_b)
            drain(rows_a, sem_a)

            @pl.when(j > 0)
            def _():
                drain_store()

            add(rows_a, 0)

            @pl.when(2 * j + 2 < nchunk)
            def _():
                stage(2 * j + 2, idx_a, rows_a, sem_a)

            drain(rows_b, sem_b)
            add(rows_b, chunk // 2)
            pltpu.async_copy(
                s_v, o_hbm.at[pl.ds(w_base // 2 + j * chunk, chunk)],
                sem_st)
            return carry

        lax.fori_loop(0, npair, pair_body, 0)
        if leftover:
            hc = chunk // 2
            drain(rows_a, sem_a)
            drain_store()
            add(rows_a, 0)
            pltpu.async_copy(
                s_v.at[pl.ds(0, hc)],
                o_hbm.at[pl.ds(w_base // 2 + npair * chunk, hc)], sem_st)
            pltpu.make_async_copy(
                s_v.at[pl.ds(0, hc)], o_hbm.at[pl.ds(0, hc)], sem_st).wait()
        else:
            drain_store()

    return gather_kernel(t1, p2, t3, i0, i1, i2)


def _tail_body(Bt, Lh, D, s_ref, ba_ref, bb_ref, aa_ref, ab_ref, o_ref):
    h = D // 2
    w3 = s_ref[...].reshape(Bt, Lh, D)
    A = pltpu.bitcast(w3 << jnp.uint32(16), jnp.float32)
    Bm = pltpu.bitcast(w3 & jnp.uint32(0xFFFF0000), jnp.float32)
    ct_a = jnp.tanh(A + ba_ref[...].reshape(1, 1, D))
    ct_b = jnp.tanh(Bm + bb_ref[...].reshape(1, 1, D))
    z = ct_a * aa_ref[...].reshape(1, 1, D) + ct_b * ab_ref[...].reshape(1, 1, D)
    le = jnp.sum(z[:, :, 0:h], axis=2, keepdims=True)
    lo = jnp.sum(z[:, :, h:D], axis=2, keepdims=True)
    m = jnp.maximum(jnp.max(le, axis=1, keepdims=True),
                    jnp.max(lo, axis=1, keepdims=True))
    ee = jnp.exp(le - m)
    eo = jnp.exp(lo - m)
    zsum = jnp.sum(ee, axis=1, keepdims=True) + jnp.sum(eo, axis=1,
                                                        keepdims=True)
    pe = ee / zsum
    po = eo / zsum
    pw = jnp.concatenate([jnp.broadcast_to(pe, (Bt, Lh, h)),
                          jnp.broadcast_to(po, (Bt, Lh, h))], axis=2)
    va = jnp.sum(ct_a * pw, axis=1)
    vb = jnp.sum(ct_b * pw, axis=1)
    o_ref[...] = jnp.concatenate([va[:, 0:h] + va[:, h:D],
                                  vb[:, 0:h] + vb[:, h:D]], axis=1)


def _tc_tail(s2, b, ap, B, L, D, Bt=16):
    Lh = L // 2
    grid = (B // Bt,)
    vblk = pl.BlockSpec((1, D), lambda i: (0, 0))
    h = D // 2
    b0, b1 = b[0:h], b[h:D]
    a0, a1 = ap[0:h], ap[h:D]
    ba = jnp.concatenate([b0, b0]).reshape(1, D)
    bb = jnp.concatenate([b1, b1]).reshape(1, D)
    aa = jnp.concatenate([a0, a0]).reshape(1, D)
    ab = jnp.concatenate([a1, a1]).reshape(1, D)
    return pl.pallas_call(
        functools.partial(_tail_body, Bt, Lh, D),
        grid=grid,
        in_specs=[
            pl.BlockSpec((Bt * Lh, D), lambda i: (i, 0)),
            vblk, vblk, vblk, vblk,
        ],
        out_specs=pl.BlockSpec((Bt, D), lambda i: (i, 0)),
        out_shape=jax.ShapeDtypeStruct((B, D), jnp.float32),
    )(s2, ba, bb, aa, ab)


def kernel(x, token_table, path_table, attn_param, W, b):
    B, L, _ = x.shape
    V, D = token_table.shape
    BL = B * L
    xf = x.reshape(BL, 3)
    half = V // 2
    xr = jnp.where(xf >= half, 2 * (xf - half) + 1, 2 * xf)
    t1, p2, t3 = _tc_transform(token_table, path_table, W)
    t1v = t1.reshape(V, D // 2)
    p2v = p2.reshape(V, D // 2)
    t3v = t3.reshape(V, D // 2)
    Bh = B // 2
    BLh = BL // 2
    ap = attn_param.reshape(D)
    outs = []
    for hx in range(2):
        lo = hx * BLh
        s = _sc_gather_sum(t1v, p2v, t3v,
                           xr[lo:lo + BLh, 0], xr[lo:lo + BLh, 1],
                           xr[lo:lo + BLh, 2])
        outs.append(_tc_tail(s, b, ap, Bh, L, D))
    return jnp.concatenate(outs, axis=0)


# speedup vs baseline: 2.1413x; 1.2119x over previous
import functools

import jax
import jax.numpy as jnp
from jax import lax
from jax.experimental import pallas as pl
from jax.experimental.pallas import tpu as pltpu
from jax.experimental.pallas import tpu_sc as plsc

_NC = 2
_NS = 16
_NW = _NC * _NS
_CHUNK = 128


def _transform_body(D, ta_ref, tb_ref, pa_ref, pb_ref, w_ref,
                    t1_ref, p2_ref, t3_ref):
    f32 = jnp.float32
    h = D // 2

    c_rnd = jnp.uint32(0x8000)

    def pack(m):
        lo = (pltpu.bitcast(m[:, 0:h], jnp.uint32) + c_rnd) >> 16
        hi = (pltpu.bitcast(m[:, h:D], jnp.uint32) + c_rnd) \
            & jnp.uint32(0xFFFF0000)
        return hi | lo

    def both(xa, xb, wslice):
        ya = pack(jnp.dot(xa, wslice, preferred_element_type=f32))
        yb = pack(jnp.dot(xb, wslice, preferred_element_type=f32))
        return jnp.concatenate([ya, yb], axis=1)

    ta = ta_ref[...]
    tb = tb_ref[...]
    pa = pa_ref[...]
    pb = pb_ref[...]
    t1_ref[...] = both(ta, tb, w_ref[0:D, :])
    p2_ref[...] = both(pa, pb, w_ref[D:2 * D, :])
    t3_ref[...] = both(ta, tb, w_ref[2 * D:3 * D, :])


def _tc_transform(token_table, path_table, W, Vt2=2000):
    V, D = token_table.shape
    half = V // 2
    nb = half // Vt2
    assert nb * Vt2 == half
    grid = (nb,)
    ablk = pl.BlockSpec((Vt2, D), lambda i: (i, 0))
    bblk = pl.BlockSpec((Vt2, D), lambda i: (i + nb, 0))
    oblk = pl.BlockSpec((Vt2, D), lambda i: (i, 0))
    out_t = jax.ShapeDtypeStruct((half, D), jnp.uint32)
    return pl.pallas_call(
        functools.partial(_transform_body, D),
        grid=grid,
        in_specs=[ablk, bblk, ablk, bblk,
                  pl.BlockSpec((3 * D, D), lambda i: (0, 0))],
        out_specs=[oblk, oblk, oblk],
        out_shape=[out_t, out_t, out_t],
    )(token_table, token_table, path_table, path_table, W)


def _sc_gather_sum(t1, p2, t3, i0, i1, i2, chunk=_CHUNK):
    (BL,) = i0.shape
    V, Dw = t1.shape
    rows_per_w = BL // _NW
    nchunk = rows_per_w // chunk
    npair = nchunk // 2
    leftover = nchunk - 2 * npair
    assert rows_per_w * _NW == BL and nchunk * chunk == rows_per_w
    ngrp = Dw // 16

    mesh = plsc.VectorSubcoreMesh(
        core_axis_name="c", subcore_axis_name="s",
        num_cores=_NC, num_subcores=_NS)

    idx_t = pltpu.VMEM((chunk,), jnp.int32)
    row_t = pltpu.VMEM((chunk, Dw), jnp.uint32)
    st_t = pltpu.VMEM((chunk, 2 * Dw), jnp.uint32)

    @functools.partial(
        pl.kernel,
        out_type=jax.ShapeDtypeStruct((BL // 2, 2 * Dw), jnp.uint32),
        mesh=mesh,
        scratch_types=[
            [idx_t] * 3, [idx_t] * 3,
            [row_t] * 3, [row_t] * 3,
            st_t,
            pltpu.SemaphoreType.DMA, pltpu.SemaphoreType.DMA,
            pltpu.SemaphoreType.DMA,
        ],
        compiler_params=pltpu.CompilerParams(use_tc_tiling_on_sc=False),
    )
    def gather_kernel(t1_hbm, p2_hbm, t3_hbm, i0_hbm, i1_hbm, i2_hbm,
                      o_hbm, idx_a, idx_b, rows_a, rows_b, s_v,
                      sem_a, sem_b, sem_st):
        wid = lax.axis_index("s") * _NC + lax.axis_index("c")
        w_base = wid * rows_per_w
        tabs = (t1_hbm, p2_hbm, t3_hbm)
        idxs = (i0_hbm, i1_hbm, i2_hbm)

        def stage(chunk_no, idx_v, rows_v, sem):
            base = w_base + chunk_no * chunk
            for t in range(3):
                pltpu.sync_copy(idxs[t].at[pl.ds(base, chunk)], idx_v[t])
            for t in range(3):
                pltpu.async_copy(tabs[t].at[idx_v[t]], rows_v[t], sem)

        def drain(rows_v, sem):
            for t in range(3):
                pltpu.make_async_copy(tabs[t].at[pl.ds(0, chunk)], rows_v[t],
                                      sem).wait()

        def drain_store():
            pltpu.make_async_copy(s_v, o_hbm.at[pl.ds(0, chunk)],
                                  sem_st).wait()

        def add(rows_v, s_off):
            r0, r1, r2 = rows_v
            c_hi = jnp.uint32(0xFFFF0000)
            c_rnd = jnp.uint32(0x8000)
            sh = jnp.uint32(16)

            @plsc.parallel_loop(0, chunk // 2)
            def pair_rows(rp):
                for half in range(2):
                    r = rp * 2 + half
                    for g in range(ngrp):
                        sl = (r, pl.ds(g * 16, 16))
                        w0, w1, w2 = r0[sl], r1[sl], r2[sl]
                        lo = (lax.bitcast_convert_type(w0 << sh, jnp.float32)
                              + lax.bitcast_convert_type(w1 << sh, jnp.float32)
                              + lax.bitcast_convert_type(w2 << sh, jnp.float32))
                        hi = (lax.bitcast_convert_type(w0 & c_hi, jnp.float32)
                              + lax.bitcast_convert_type(w1 & c_hi, jnp.float32)
                              + lax.bitcast_convert_type(w2 & c_hi, jnp.float32))
                        lo_r = (lax.bitcast_convert_type(lo, jnp.uint32)
                                + c_rnd) >> sh
                        hi_r = (lax.bitcast_convert_type(hi, jnp.uint32)
                                + c_rnd) & c_hi
                        s_v[s_off + rp, pl.ds(half * Dw + g * 16, 16)] = \
                            hi_r | lo_r

        stage(0, idx_a, rows_a, sem_a)

        def pair_body(j, carry):
            stage(2 * j + 1, idx_b, rows_b, sem_b)
            drain(rows_a, sem_a)

            @pl.when(j > 0)
            def _():
                drain_store()

            add(rows_a, 0)

            @pl.when(2 * j + 2 < nchunk)
            def _():
                stage(2 * j + 2, idx_a, rows_a, sem_a)

            drain(rows_b, sem_b)
            add(rows_b, chunk // 2)
            pltpu.async_copy(
                s_v, o_hbm.at[pl.ds(w_base // 2 + j * chunk, chunk)],
                sem_st)
            return carry

        lax.fori_loop(0, npair, pair_body, 0)
        if leftover:
            hc = chunk // 2
            drain(rows_a, sem_a)
            drain_store()
            add(rows_a, 0)
            pltpu.async_copy(
                s_v.at[pl.ds(0, hc)],
                o_hbm.at[pl.ds(w_base // 2 + npair * chunk, hc)], sem_st)
            pltpu.make_async_copy(
                s_v.at[pl.ds(0, hc)], o_hbm.at[pl.ds(0, hc)], sem_st).wait()
        else:
            drain_store()

    return gather_kernel(t1, p2, t3, i0, i1, i2)


def _tail_body(Bt, Lh, D, s_ref, ba_ref, bb_ref, aa_ref, ab_ref, o_ref):
    h = D // 2
    w3 = s_ref[...].reshape(Bt, Lh, D)
    A = pltpu.bitcast(w3 << jnp.uint32(16), jnp.float32)
    Bm = pltpu.bitcast(w3 & jnp.uint32(0xFFFF0000), jnp.float32)
    ct_a = jnp.tanh(A + ba_ref[...].reshape(1, 1, D))
    ct_b = jnp.tanh(Bm + bb_ref[...].reshape(1, 1, D))
    z = ct_a * aa_ref[...].reshape(1, 1, D) + ct_b * ab_ref[...].reshape(1, 1, D)
    le = jnp.sum(z[:, :, 0:h], axis=2, keepdims=True)
    lo = jnp.sum(z[:, :, h:D], axis=2, keepdims=True)
    m = jnp.maximum(jnp.max(le, axis=1, keepdims=True),
                    jnp.max(lo, axis=1, keepdims=True))
    ee = jnp.exp(le - m)
    eo = jnp.exp(lo - m)
    zsum = jnp.sum(ee, axis=1, keepdims=True) + jnp.sum(eo, axis=1,
                                                        keepdims=True)
    pe = ee / zsum
    po = eo / zsum
    pw = jnp.concatenate([jnp.broadcast_to(pe, (Bt, Lh, h)),
                          jnp.broadcast_to(po, (Bt, Lh, h))], axis=2)
    va = jnp.sum(ct_a * pw, axis=1)
    vb = jnp.sum(ct_b * pw, axis=1)
    o_ref[...] = jnp.concatenate([va[:, 0:h] + va[:, h:D],
                                  vb[:, 0:h] + vb[:, h:D]], axis=1)


def _tc_tail(s2, b, ap, B, L, D, Bt=16):
    Lh = L // 2
    grid = (B // Bt,)
    vblk = pl.BlockSpec((1, D), lambda i: (0, 0))
    h = D // 2
    b0, b1 = b[0:h], b[h:D]
    a0, a1 = ap[0:h], ap[h:D]
    ba = jnp.concatenate([b0, b0]).reshape(1, D)
    bb = jnp.concatenate([b1, b1]).reshape(1, D)
    aa = jnp.concatenate([a0, a0]).reshape(1, D)
    ab = jnp.concatenate([a1, a1]).reshape(1, D)
    return pl.pallas_call(
        functools.partial(_tail_body, Bt, Lh, D),
        grid=grid,
        in_specs=[
            pl.BlockSpec((Bt * Lh, D), lambda i: (i, 0)),
            vblk, vblk, vblk, vblk,
        ],
        out_specs=pl.BlockSpec((Bt, D), lambda i: (i, 0)),
        out_shape=jax.ShapeDtypeStruct((B, D), jnp.float32),
    )(s2, ba, bb, aa, ab)


def kernel(x, token_table, path_table, attn_param, W, b):
    B, L, _ = x.shape
    V, D = token_table.shape
    BL = B * L
    xf = x.reshape(BL, 3)
    half = V // 2
    xr = jnp.where(xf >= half, 2 * (xf - half) + 1, 2 * xf)
    t1, p2, t3 = _tc_transform(token_table, path_table, W)
    t1v = t1.reshape(V, D // 2)
    p2v = p2.reshape(V, D // 2)
    t3v = t3.reshape(V, D // 2)
    Bh = B // 2
    BLh = BL // 2
    ap = attn_param.reshape(D)
    outs = []
    for hx in range(2):
        lo = hx * BLh
        s = _sc_gather_sum(t1v, p2v, t3v,
                           xr[lo:lo + BLh, 0], xr[lo:lo + BLh, 1],
                           xr[lo:lo + BLh, 2])
        outs.append(_tc_tail(s, b, ap, Bh, L, D))
    return jnp.concatenate(outs, axis=0)


# speedup vs baseline: 2.1774x; 1.0169x over previous
import functools

import jax
import jax.numpy as jnp
from jax import lax
from jax.experimental import pallas as pl
from jax.experimental.pallas import tpu as pltpu
from jax.experimental.pallas import tpu_sc as plsc

_NC = 2
_NS = 16
_NW = _NC * _NS
_CHUNK = 128


def _transform_body(D, ta_ref, tb_ref, pa_ref, pb_ref, w_ref,
                    t1_ref, p2_ref, t3_ref):
    f32 = jnp.float32
    h = D // 2

    c_rnd = jnp.uint32(0x8000)

    def pack(m):
        lo = (pltpu.bitcast(m[:, 0:h], jnp.uint32) + c_rnd) >> 16
        hi = (pltpu.bitcast(m[:, h:D], jnp.uint32) + c_rnd) \
            & jnp.uint32(0xFFFF0000)
        return hi | lo

    def both(xa, xb, wslice):
        ya = pack(jnp.dot(xa, wslice, preferred_element_type=f32))
        yb = pack(jnp.dot(xb, wslice, preferred_element_type=f32))
        return jnp.concatenate([ya, yb], axis=1)

    ta = ta_ref[...]
    tb = tb_ref[...]
    pa = pa_ref[...]
    pb = pb_ref[...]
    t1_ref[...] = both(ta, tb, w_ref[0:D, :])
    p2_ref[...] = both(pa, pb, w_ref[D:2 * D, :])
    t3_ref[...] = both(ta, tb, w_ref[2 * D:3 * D, :])


def _tc_transform(token_table, path_table, W, Vt2=5000):
    V, D = token_table.shape
    half = V // 2
    nb = half // Vt2
    assert nb * Vt2 == half
    grid = (nb,)
    ablk = pl.BlockSpec((Vt2, D), lambda i: (i, 0))
    bblk = pl.BlockSpec((Vt2, D), lambda i: (i + nb, 0))
    oblk = pl.BlockSpec((Vt2, D), lambda i: (i, 0))
    out_t = jax.ShapeDtypeStruct((half, D), jnp.uint32)
    return pl.pallas_call(
        functools.partial(_transform_body, D),
        grid=grid,
        in_specs=[ablk, bblk, ablk, bblk,
                  pl.BlockSpec((3 * D, D), lambda i: (0, 0))],
        out_specs=[oblk, oblk, oblk],
        out_shape=[out_t, out_t, out_t],
    )(token_table, token_table, path_table, path_table, W)


def _sc_gather_sum(t1, p2, t3, i0, i1, i2, chunk=_CHUNK):
    (BL,) = i0.shape
    V, Dw = t1.shape
    rows_per_w = BL // _NW
    nchunk = rows_per_w // chunk
    npair = nchunk // 2
    leftover = nchunk - 2 * npair
    assert rows_per_w * _NW == BL and nchunk * chunk == rows_per_w
    ngrp = Dw // 16

    mesh = plsc.VectorSubcoreMesh(
        core_axis_name="c", subcore_axis_name="s",
        num_cores=_NC, num_subcores=_NS)

    idx_t = pltpu.VMEM((chunk,), jnp.int32)
    row_t = pltpu.VMEM((chunk, Dw), jnp.uint32)
    st_t = pltpu.VMEM((chunk, 2 * Dw), jnp.uint32)

    @functools.partial(
        pl.kernel,
        out_type=jax.ShapeDtypeStruct((BL // 2, 2 * Dw), jnp.uint32),
        mesh=mesh,
        scratch_types=[
            [idx_t] * 3, [idx_t] * 3,
            [row_t] * 3, [row_t] * 3,
            st_t,
            pltpu.SemaphoreType.DMA, pltpu.SemaphoreType.DMA,
            pltpu.SemaphoreType.DMA,
        ],
        compiler_params=pltpu.CompilerParams(use_tc_tiling_on_sc=False),
    )
    def gather_kernel(t1_hbm, p2_hbm, t3_hbm, i0_hbm, i1_hbm, i2_hbm,
                      o_hbm, idx_a, idx_b, rows_a, rows_b, s_v,
                      sem_a, sem_b, sem_st):
        wid = lax.axis_index("s") * _NC + lax.axis_index("c")
        w_base = wid * rows_per_w
        tabs = (t1_hbm, p2_hbm, t3_hbm)
        idxs = (i0_hbm, i1_hbm, i2_hbm)

        def stage(chunk_no, idx_v, rows_v, sem):
            base = w_base + chunk_no * chunk
            for t in range(3):
                pltpu.sync_copy(idxs[t].at[pl.ds(base, chunk)], idx_v[t])
            for t in range(3):
                pltpu.async_copy(tabs[t].at[idx_v[t]], rows_v[t], sem)

        def drain(rows_v, sem):
            for t in range(3):
                pltpu.make_async_copy(tabs[t].at[pl.ds(0, chunk)], rows_v[t],
                                      sem).wait()

        def drain_store():
            pltpu.make_async_copy(s_v, o_hbm.at[pl.ds(0, chunk)],
                                  sem_st).wait()

        def add(rows_v, s_off):
            r0, r1, r2 = rows_v
            c_hi = jnp.uint32(0xFFFF0000)
            c_rnd = jnp.uint32(0x8000)
            sh = jnp.uint32(16)

            @plsc.parallel_loop(0, chunk // 2)
            def pair_rows(rp):
                for half in range(2):
                    r = rp * 2 + half
                    for g in range(ngrp):
                        sl = (r, pl.ds(g * 16, 16))
                        w0, w1, w2 = r0[sl], r1[sl], r2[sl]
                        lo = (lax.bitcast_convert_type(w0 << sh, jnp.float32)
                              + lax.bitcast_convert_type(w1 << sh, jnp.float32)
                              + lax.bitcast_convert_type(w2 << sh, jnp.float32))
                        hi = (lax.bitcast_convert_type(w0 & c_hi, jnp.float32)
                              + lax.bitcast_convert_type(w1 & c_hi, jnp.float32)
                              + lax.bitcast_convert_type(w2 & c_hi, jnp.float32))
                        lo_r = (lax.bitcast_convert_type(lo, jnp.uint32)
                                + c_rnd) >> sh
                        hi_r = (lax.bitcast_convert_type(hi, jnp.uint32)
                                + c_rnd) & c_hi
                        s_v[s_off + rp, pl.ds(half * Dw + g * 16, 16)] = \
                            hi_r | lo_r

        stage(0, idx_a, rows_a, sem_a)

        def pair_body(j, carry):
            stage(2 * j + 1, idx_b, rows_b, sem_b)
            drain(rows_a, sem_a)

            @pl.when(j > 0)
            def _():
                drain_store()

            add(rows_a, 0)

            @pl.when(2 * j + 2 < nchunk)
            def _():
                stage(2 * j + 2, idx_a, rows_a, sem_a)

            drain(rows_b, sem_b)
            add(rows_b, chunk // 2)
            pltpu.async_copy(
                s_v, o_hbm.at[pl.ds(w_base // 2 + j * chunk, chunk)],
                sem_st)
            return carry

        lax.fori_loop(0, npair, pair_body, 0)
        if leftover:
            hc = chunk // 2
            drain(rows_a, sem_a)
            drain_store()
            add(rows_a, 0)
            pltpu.async_copy(
                s_v.at[pl.ds(0, hc)],
                o_hbm.at[pl.ds(w_base // 2 + npair * chunk, hc)], sem_st)
            pltpu.make_async_copy(
                s_v.at[pl.ds(0, hc)], o_hbm.at[pl.ds(0, hc)], sem_st).wait()
        else:
            drain_store()

    return gather_kernel(t1, p2, t3, i0, i1, i2)


def _tail_body(Bt, Lh, D, s_ref, ba_ref, bb_ref, aa_ref, ab_ref, o_ref):
    h = D // 2
    w3 = s_ref[...].reshape(Bt, Lh, D)
    A = pltpu.bitcast(w3 << jnp.uint32(16), jnp.float32)
    Bm = pltpu.bitcast(w3 & jnp.uint32(0xFFFF0000), jnp.float32)
    ct_a = jnp.tanh(A + ba_ref[...].reshape(1, 1, D))
    ct_b = jnp.tanh(Bm + bb_ref[...].reshape(1, 1, D))
    z = ct_a * aa_ref[...].reshape(1, 1, D) + ct_b * ab_ref[...].reshape(1, 1, D)
    le = jnp.sum(z[:, :, 0:h], axis=2, keepdims=True)
    lo = jnp.sum(z[:, :, h:D], axis=2, keepdims=True)
    m = jnp.maximum(jnp.max(le, axis=1, keepdims=True),
                    jnp.max(lo, axis=1, keepdims=True))
    ee = jnp.exp(le - m)
    eo = jnp.exp(lo - m)
    zsum = jnp.sum(ee, axis=1, keepdims=True) + jnp.sum(eo, axis=1,
                                                        keepdims=True)
    pe = ee / zsum
    po = eo / zsum
    pw = jnp.concatenate([jnp.broadcast_to(pe, (Bt, Lh, h)),
                          jnp.broadcast_to(po, (Bt, Lh, h))], axis=2)
    va = jnp.sum(ct_a * pw, axis=1)
    vb = jnp.sum(ct_b * pw, axis=1)
    o_ref[...] = jnp.concatenate([va[:, 0:h] + va[:, h:D],
                                  vb[:, 0:h] + vb[:, h:D]], axis=1)


def _tc_tail(s2, b, ap, B, L, D, Bt=16):
    Lh = L // 2
    grid = (B // Bt,)
    vblk = pl.BlockSpec((1, D), lambda i: (0, 0))
    h = D // 2
    b0, b1 = b[0:h], b[h:D]
    a0, a1 = ap[0:h], ap[h:D]
    ba = jnp.concatenate([b0, b0]).reshape(1, D)
    bb = jnp.concatenate([b1, b1]).reshape(1, D)
    aa = jnp.concatenate([a0, a0]).reshape(1, D)
    ab = jnp.concatenate([a1, a1]).reshape(1, D)
    return pl.pallas_call(
        functools.partial(_tail_body, Bt, Lh, D),
        grid=grid,
        in_specs=[
            pl.BlockSpec((Bt * Lh, D), lambda i: (i, 0)),
            vblk, vblk, vblk, vblk,
        ],
        out_specs=pl.BlockSpec((Bt, D), lambda i: (i, 0)),
        out_shape=jax.ShapeDtypeStruct((B, D), jnp.float32),
    )(s2, ba, bb, aa, ab)


def kernel(x, token_table, path_table, attn_param, W, b):
    B, L, _ = x.shape
    V, D = token_table.shape
    BL = B * L
    xf = x.reshape(BL, 3)
    half = V // 2
    xr = jnp.where(xf >= half, 2 * (xf - half) + 1, 2 * xf)
    t1, p2, t3 = _tc_transform(token_table, path_table, W)
    t1v = t1.reshape(V, D // 2)
    p2v = p2.reshape(V, D // 2)
    t3v = t3.reshape(V, D // 2)
    Bh = B // 2
    BLh = BL // 2
    ap = attn_param.reshape(D)
    outs = []
    for hx in range(2):
        lo = hx * BLh
        s = _sc_gather_sum(t1v, p2v, t3v,
                           xr[lo:lo + BLh, 0], xr[lo:lo + BLh, 1],
                           xr[lo:lo + BLh, 2])
        outs.append(_tc_tail(s, b, ap, Bh, L, D))
    return jnp.concatenate(outs, axis=0)
